# Initial kernel scaffold; baseline (speedup 1.0000x reference)
import jax
import jax.numpy as jnp
from jax.experimental import pallas as pl


def kernel(x, edge_index, conv_W0, conv_b0, conv_W1, conv_b1, conv_W2, conv_b2, bn_g0, bn_b0, bn_g1, bn_b1, bn_g2, bn_b2, pool_W, pool_b, c1_W, c1_b, c2_W, c2_b, fc1_W, fc1_b, fc2_W, fc2_b):
    raise NotImplementedError("write your pallas kernel here")


# speedup vs baseline: 6.6844x; 6.6844x over previous
import functools

import jax
import jax.numpy as jnp
from jax import lax
from jax.experimental import pallas as pl
from jax.experimental.pallas import tpu as pltpu
from jax.experimental.pallas import tpu_sc as plsc

f32 = jnp.float32
i32 = jnp.int32

N = 10000
NPAD = 10240
E = 320000
EPAD = 327680
EROWS = EPAD // 128
CH_FULL = EROWS // 16
CH_HALF = EROWS // 32
RPT = NPAD // 16
DH = 64
K = 5000

@functools.lru_cache(maxsize=1)
def _mesh():
    return plsc.VectorSubcoreMesh(core_axis_name="c", subcore_axis_name="s")


def _zero_rows(buf, width):
    @pl.loop(0, 128)
    def _(r):
        @pl.loop(0, width, step=16)
        def _(t):
            buf[r, pl.ds(t, 16)] = jnp.zeros((16,), f32)


def _sc_deg_body(src_hbm, dst_hbm, dego_hbm, degi_hbm, idxv, buf, acc, sem):
    cid = lax.axis_index("c")
    sid = lax.axis_index("s")

    @pl.when(cid == 0)
    def _():
        pltpu.sync_copy(src_hbm.at[pl.ds(sid * jnp.int32(CH_FULL), CH_FULL)], idxv)

    @pl.when(cid == 1)
    def _():
        pltpu.sync_copy(dst_hbm.at[pl.ds(sid * jnp.int32(CH_FULL), CH_FULL)], idxv)

    _zero_rows(buf, 16)
    @pl.loop(0, RPT // 128)
    def _(r):
        pltpu.sync_copy(buf, acc.at[pl.ds(sid * jnp.int32(RPT) + r * jnp.int32(128), 128)])

    @pl.loop(0, 128)
    def _(r):
        buf[r, pl.ds(0, 16)] = jnp.full((16,), 1.0, f32)

    plsc.subcore_barrier()

    @pl.loop(0, CH_FULL)
    def _(j):
        pltpu.sync_copy(buf, acc.at[idxv.at[j]], add=True)

    plsc.subcore_barrier()

    @pl.when(cid == 0)
    def _():
        pltpu.sync_copy(acc.at[pl.ds(sid * jnp.int32(RPT), RPT)],
                        dego_hbm.at[pl.ds(sid * jnp.int32(RPT), RPT)])

    @pl.when(cid == 1)
    def _():
        pltpu.sync_copy(acc.at[pl.ds(sid * jnp.int32(RPT), RPT)],
                        degi_hbm.at[pl.ds(sid * jnp.int32(RPT), RPT)])


@functools.lru_cache(maxsize=1)
def _sc_deg():
    return pl.kernel(
        _sc_deg_body,
        mesh=_mesh(),
        out_type=[jax.ShapeDtypeStruct((NPAD, 16), f32),
                  jax.ShapeDtypeStruct((NPAD, 16), f32)],
        scratch_types=[
            pltpu.VMEM((CH_FULL, 128), i32),
            pltpu.VMEM((128, 16), f32),
            pltpu.VMEM_SHARED((NPAD, 16), f32),
            pltpu.SemaphoreType.DMA,
        ],
        compiler_params=pltpu.CompilerParams(use_tc_tiling_on_sc=False),
    )


def _sc_agg_body(ha, hb, src_hbm, dst_hbm, outa, outb, srcv, dstv, buf, acc,
                 sem):
    cid = lax.axis_index("c")
    sid = lax.axis_index("s")

    pltpu.sync_copy(src_hbm.at[pl.ds(sid * jnp.int32(CH_FULL), CH_FULL)], srcv)
    pltpu.sync_copy(dst_hbm.at[pl.ds(sid * jnp.int32(CH_FULL), CH_FULL)], dstv)

    _zero_rows(buf, DH)
    @pl.loop(0, RPT // 128)
    def _(r):
        pltpu.sync_copy(buf, acc.at[pl.ds(sid * jnp.int32(RPT) + r * jnp.int32(128), 128)])

    plsc.subcore_barrier()

    def run(h_ref):
        @pl.loop(0, CH_FULL)
        def _(j):
            pltpu.async_copy(h_ref.at[srcv.at[j]], buf, sem).wait()
            pltpu.sync_copy(buf, acc.at[dstv.at[j]], add=True)

    @pl.when(cid == 0)
    def _():
        run(ha)

    @pl.when(cid == 1)
    def _():
        run(hb)

    plsc.subcore_barrier()

    @pl.when(cid == 0)
    def _():
        pltpu.sync_copy(acc.at[pl.ds(sid * jnp.int32(RPT), RPT)],
                        outa.at[pl.ds(sid * jnp.int32(RPT), RPT)])

    @pl.when(cid == 1)
    def _():
        pltpu.sync_copy(acc.at[pl.ds(sid * jnp.int32(RPT), RPT)],
                        outb.at[pl.ds(sid * jnp.int32(RPT), RPT)])


@functools.lru_cache(maxsize=1)
def _sc_agg():
    return pl.kernel(
        _sc_agg_body,
        mesh=_mesh(),
        out_type=[jax.ShapeDtypeStruct((NPAD, DH), f32),
                  jax.ShapeDtypeStruct((NPAD, DH), f32)],
        scratch_types=[
            pltpu.VMEM((CH_FULL, 128), i32),
            pltpu.VMEM((CH_FULL, 128), i32),
            pltpu.VMEM((128, DH), f32),
            pltpu.VMEM_SHARED((NPAD, DH), f32),
            pltpu.SemaphoreType.DMA,
        ],
        compiler_params=pltpu.CompilerParams(use_tc_tiling_on_sc=False),
    )


def _sc_score_body(spre, src_hbm, dst_hbm, outa, outb, srcv, dstv, buf, acc,
                   sem):
    cid = lax.axis_index("c")
    sid = lax.axis_index("s")
    base = cid * jnp.int32(EROWS // 2) + sid * jnp.int32(CH_HALF)

    pltpu.sync_copy(src_hbm.at[pl.ds(base, CH_HALF)], srcv)
    pltpu.sync_copy(dst_hbm.at[pl.ds(base, CH_HALF)], dstv)

    _zero_rows(buf, 16)
    @pl.loop(0, RPT // 128)
    def _(r):
        pltpu.sync_copy(buf, acc.at[pl.ds(sid * jnp.int32(RPT) + r * jnp.int32(128), 128)])

    plsc.subcore_barrier()

    @pl.loop(0, CH_HALF)
    def _(j):
        pltpu.async_copy(spre.at[srcv.at[j]], buf, sem).wait()
        pltpu.sync_copy(buf, acc.at[dstv.at[j]], add=True)

    plsc.subcore_barrier()

    @pl.when(cid == 0)
    def _():
        pltpu.sync_copy(acc.at[pl.ds(sid * jnp.int32(RPT), RPT)],
                        outa.at[pl.ds(sid * jnp.int32(RPT), RPT)])

    @pl.when(cid == 1)
    def _():
        pltpu.sync_copy(acc.at[pl.ds(sid * jnp.int32(RPT), RPT)],
                        outb.at[pl.ds(sid * jnp.int32(RPT), RPT)])


@functools.lru_cache(maxsize=1)
def _sc_score():
    return pl.kernel(
        _sc_score_body,
        mesh=_mesh(),
        out_type=[jax.ShapeDtypeStruct((NPAD, 16), f32),
                  jax.ShapeDtypeStruct((NPAD, 16), f32)],
        scratch_types=[
            pltpu.VMEM((CH_HALF, 128), i32),
            pltpu.VMEM((CH_HALF, 128), i32),
            pltpu.VMEM((128, 16), f32),
            pltpu.VMEM_SHARED((NPAD, 16), f32),
            pltpu.SemaphoreType.DMA,
        ],
        compiler_params=pltpu.CompilerParams(use_tc_tiling_on_sc=False),
    )


_HP = dict(preferred_element_type=f32, precision=lax.Precision.HIGHEST)


def _tc_pre(x_ref, dego_ref, degi_ref, h0a_ref, h0b_ref, dout_ref, din_ref):
    dout = lax.rsqrt(jnp.maximum(dego_ref[:, 0:1], 1.0))
    din = lax.rsqrt(jnp.maximum(degi_ref[:, 0:1], 1.0))
    h0 = x_ref[...] * dout
    h0a_ref[...] = h0[:, :DH]
    h0b_ref[...] = h0[:, DH:]
    dout_ref[...] = dout
    din_ref[...] = din


def _layer_core(agga_ref, aggb_ref, din_ref, w_ref, b_ref, g_ref, bb_ref):
    agg = jnp.concatenate([agga_ref[...], aggb_ref[...]], axis=1)
    agg = agg * din_ref[...]
    z = jnp.dot(agg, w_ref[...], **_HP) + b_ref[...]
    mask = (lax.broadcasted_iota(i32, (NPAD, 1), 0) < N).astype(f32)
    m = jnp.sum(z * mask, axis=0, keepdims=True) / N
    ex2 = jnp.sum(z * z * mask, axis=0, keepdims=True) / N
    v = ex2 - m * m
    feat = (z - m) * lax.rsqrt(v + 1e-5) * g_ref[...] + bb_ref[...]
    return jnp.maximum(feat, 0.0)


def _tc_layer(agga_ref, aggb_ref, din_ref, dout_ref, w_ref, b_ref, g_ref,
              bb_ref, ha_ref, hb_ref):
    feat = _layer_core(agga_ref, aggb_ref, din_ref, w_ref, b_ref, g_ref,
                       bb_ref)
    h = feat * dout_ref[...]
    ha_ref[...] = h[:, :DH]
    hb_ref[...] = h[:, DH:]


def _tc_layer3(agga_ref, aggb_ref, din_ref, dout_ref, w_ref, b_ref, g_ref,
               bb_ref, pw_ref, feat_ref, spre_ref):
    feat = _layer_core(agga_ref, aggb_ref, din_ref, w_ref, b_ref, g_ref,
                       bb_ref)
    feat_ref[...] = feat
    h = feat * dout_ref[...]
    spre = jnp.sum(h * pw_ref[...], axis=1, keepdims=True)
    spre_ref[...] = jnp.broadcast_to(spre, (NPAD, 16))


def _tc_top(sca_ref, scb_ref, din_ref, pb_ref, score_ref):
    s = (sca_ref[:, 0:1] + scb_ref[:, 0:1]) * din_ref[...] + pb_ref[0, 0]
    idx = lax.broadcasted_iota(i32, (NPAD, 1), 0)
    score_ref[...] = jnp.where(idx < N, s, -jnp.inf)


def _tc_sel(s_ref, w_ref, m_ref):
    s = s_ref[...]
    b = lax.bitcast_convert_type(s, i32)
    u = jnp.where(b >= 0, b ^ jnp.int32(-2147483648), ~b)

    def vbody(t, carry):
        kk, sel, match = carry
        bit = (31 - t).astype(i32)
        bitv = lax.shift_right_logical(u, bit) & 1
        hi = match * bitv
        c1 = jnp.sum(hi.astype(f32))
        take1 = kk <= c1
        sel = jnp.where(take1, sel, sel | hi)
        kk = jnp.where(take1, kk, kk - c1)
        match = match * jnp.where(take1, bitv, 1 - bitv)
        return kk, sel, match

    init = (jnp.float32(K), jnp.zeros(s.shape, i32),
            jnp.ones(s.shape, i32))
    kk, sel, match = lax.fori_loop(0, 32, vbody, init)

    gidx = (lax.broadcasted_iota(i32, s.shape, 0) * s.shape[1]
            + lax.broadcasted_iota(i32, s.shape, 1))

    def ibody(t, carry):
        kk, sel, match = carry
        bit = (13 - t).astype(i32)
        bitv = lax.shift_right_logical(gidx, bit) & 1
        lo = match * (1 - bitv)
        c0 = jnp.sum(lo.astype(f32))
        take0 = kk <= c0
        sel = jnp.where(take0, sel, sel | lo)
        kk = jnp.where(take0, kk, kk - c0)
        match = match * jnp.where(take0, 1 - bitv, bitv)
        return kk, sel, match

    kk, sel, match = lax.fori_loop(0, 14, ibody, (kk, sel, match))
    sel = sel | match
    mf = sel.astype(f32)
    m_ref[...] = mf
    w_ref[...] = mf * jnp.tanh(s)


def _tc_read(feat_ref, w_ref, m_ref, c1a_ref, c1b_ref, c1bias_ref, c2w_ref,
             c2b_ref, f1w_ref, f1b_ref, f2w_ref, f2b_ref, out_ref):
    feat = feat_ref[...]
    w = w_ref[...]
    m = m_ref[...]
    fw = feat * w
    avg = jnp.sum(fw, axis=0, keepdims=True) / K
    mx = jnp.max(jnp.where(m > 0.5, fw, -jnp.inf), axis=0, keepdims=True)
    h = (jnp.dot(avg, c1a_ref[...], **_HP) + jnp.dot(mx, c1b_ref[...], **_HP)
         + c1bias_ref[...])
    h = jnp.maximum(h, 0.0)
    h = jnp.maximum(jnp.dot(h, c2w_ref[...], **_HP) + c2b_ref[...], 0.0)
    h = jnp.maximum(jnp.dot(h, f1w_ref[...], **_HP) + f1b_ref[...], 0.0)
    o = jnp.dot(h, f2w_ref[...], **_HP) + f2b_ref[...]
    omax = jnp.max(o, axis=1, keepdims=True)
    lse = jnp.log(jnp.sum(jnp.exp(o - omax), axis=1, keepdims=True)) + omax
    out_ref[...] = o - lse


def _sd(shape):
    return jax.ShapeDtypeStruct(shape, f32)


def _call(body, out_shapes, *args):
    return pl.pallas_call(body, out_shape=out_shapes)(*args)


def kernel(*args):
    with jax.enable_x64(False):
        return _kernel32(*args)


def _kernel32(x, edge_index, conv_W0, conv_b0, conv_W1, conv_b1, conv_W2,
              conv_b2, bn_g0, bn_b0, bn_g1, bn_b1, bn_g2, bn_b2, pool_W,
              pool_b, c1_W, c1_b, c2_W, c2_b, fc1_W, fc1_b, fc2_W, fc2_b):
    src = edge_index[0].astype(i32)
    dst = edge_index[1].astype(i32)
    pad_idx = N + (jnp.arange(EPAD - E, dtype=i32) % (NPAD - N))
    src2d = jnp.concatenate([src, pad_idx]).reshape(EROWS, 128)
    dst2d = jnp.concatenate([dst, pad_idx]).reshape(EROWS, 128)
    x_pad = jnp.concatenate([x, jnp.zeros((NPAD - N, 128), f32)], axis=0)

    b0 = conv_b0.reshape(1, 128)
    b1 = conv_b1.reshape(1, 128)
    b2 = conv_b2.reshape(1, 128)
    g0 = bn_g0.reshape(1, 128)
    g1 = bn_g1.reshape(1, 128)
    g2 = bn_g2.reshape(1, 128)
    bb0 = bn_b0.reshape(1, 128)
    bb1 = bn_b1.reshape(1, 128)
    bb2 = bn_b2.reshape(1, 128)
    pw = pool_W.reshape(1, 128)
    pb = pool_b.reshape(1, 1)
    c1a = c1_W[:128]
    c1b = c1_W[128:]
    c1bias = c1_b.reshape(1, 16)
    c2b = c2_b.reshape(1, 32)
    f1b = fc1_b.reshape(1, 128)
    f2b = fc2_b.reshape(1, 2)

    dego, degi = _sc_deg()(src2d, dst2d)
    h0a, h0b, dout, din = _call(
        _tc_pre,
        [_sd((NPAD, DH)), _sd((NPAD, DH)), _sd((NPAD, 1)), _sd((NPAD, 1))],
        x_pad, dego, degi)

    agga, aggb = _sc_agg()(h0a, h0b, src2d, dst2d)
    h1a, h1b = _call(_tc_layer, [_sd((NPAD, DH)), _sd((NPAD, DH))],
                     agga, aggb, din, dout, conv_W0, b0, g0, bb0)

    agga, aggb = _sc_agg()(h1a, h1b, src2d, dst2d)
    h2a, h2b = _call(_tc_layer, [_sd((NPAD, DH)), _sd((NPAD, DH))],
                     agga, aggb, din, dout, conv_W1, b1, g1, bb1)

    agga, aggb = _sc_agg()(h2a, h2b, src2d, dst2d)
    feat3, spre = _call(_tc_layer3, [_sd((NPAD, 128)), _sd((NPAD, 16))],
                        agga, aggb, din, dout, conv_W2, b2, g2, bb2, pw)

    sca, scb = _sc_score()(spre, src2d, dst2d)
    score = _call(_tc_top, _sd((NPAD, 1)), sca, scb, din, pb)

    w80, m80 = _call(_tc_sel, [_sd((NPAD // 128, 128))] * 2,
                     score.reshape(NPAD // 128, 128))

    out = _call(_tc_read, _sd((1, 2)), feat3, w80.reshape(NPAD, 1),
                m80.reshape(NPAD, 1), c1a, c1b, c1bias, c2_W, c2b, fc1_W,
                f1b, fc2_W, f2b)
    return out


# speedup vs baseline: 11.0798x; 1.6576x over previous
import functools

import jax
import jax.numpy as jnp
from jax import lax
from jax.experimental import pallas as pl
from jax.experimental.pallas import tpu as pltpu
from jax.experimental.pallas import tpu_sc as plsc

f32 = jnp.float32
i32 = jnp.int32

N = 10000
NPAD = 10240
E = 320000
EPAD = 327680
EROWS = EPAD // 128
CH_FULL = EROWS // 16
CH_HALF = EROWS // 32
RPT = NPAD // 16
DH = 64
K = 5000

@functools.lru_cache(maxsize=1)
def _mesh():
    return plsc.VectorSubcoreMesh(core_axis_name="c", subcore_axis_name="s")


def _zero_rows(buf, width):
    @pl.loop(0, 128)
    def _(r):
        @pl.loop(0, width, step=16)
        def _(t):
            buf[r, pl.ds(t, 16)] = jnp.zeros((16,), f32)


def _sc_deg_body(src_hbm, dst_hbm, dego_hbm, degi_hbm, idxv, buf, acc, sem):
    cid = lax.axis_index("c")
    sid = lax.axis_index("s")

    @pl.when(cid == 0)
    def _():
        pltpu.sync_copy(src_hbm.at[pl.ds(sid * jnp.int32(CH_FULL), CH_FULL)], idxv)

    @pl.when(cid == 1)
    def _():
        pltpu.sync_copy(dst_hbm.at[pl.ds(sid * jnp.int32(CH_FULL), CH_FULL)], idxv)

    _zero_rows(buf, 16)
    @pl.loop(0, RPT // 128)
    def _(r):
        pltpu.sync_copy(buf, acc.at[pl.ds(sid * jnp.int32(RPT) + r * jnp.int32(128), 128)])

    @pl.loop(0, 128)
    def _(r):
        buf[r, pl.ds(0, 16)] = jnp.full((16,), 1.0, f32)

    plsc.subcore_barrier()

    @pl.loop(0, CH_FULL)
    def _(j):
        pltpu.sync_copy(buf, acc.at[idxv.at[j]], add=True)

    plsc.subcore_barrier()

    @pl.when(cid == 0)
    def _():
        pltpu.sync_copy(acc.at[pl.ds(sid * jnp.int32(RPT), RPT)],
                        dego_hbm.at[pl.ds(sid * jnp.int32(RPT), RPT)])

    @pl.when(cid == 1)
    def _():
        pltpu.sync_copy(acc.at[pl.ds(sid * jnp.int32(RPT), RPT)],
                        degi_hbm.at[pl.ds(sid * jnp.int32(RPT), RPT)])


@functools.lru_cache(maxsize=1)
def _sc_deg():
    return pl.kernel(
        _sc_deg_body,
        mesh=_mesh(),
        out_type=[jax.ShapeDtypeStruct((NPAD, 16), f32),
                  jax.ShapeDtypeStruct((NPAD, 16), f32)],
        scratch_types=[
            pltpu.VMEM((CH_FULL, 128), i32),
            pltpu.VMEM((128, 16), f32),
            pltpu.VMEM_SHARED((NPAD, 16), f32),
            pltpu.SemaphoreType.DMA,
        ],
        compiler_params=pltpu.CompilerParams(use_tc_tiling_on_sc=False),
    )


NBUF = 4


def _ring(h_ref, acc, srcv, dstv, bufs, gsem, ssem, nch, rows):
    for b in range(NBUF):
        pltpu.async_copy(h_ref.at[srcv.at[b]], bufs[b], gsem[b])

    @pl.loop(0, nch, step=NBUF)
    def _(j):
        for b in range(NBUF):
            pltpu.make_async_copy(h_ref.at[pl.ds(0, rows)], bufs[b],
                                  gsem[b]).wait()
            pltpu.async_copy(bufs[b], acc.at[dstv.at[j + b]], ssem[b],
                             add=True)
        for b in range(NBUF):
            pltpu.make_async_copy(bufs[b], acc.at[pl.ds(0, rows)],
                                  ssem[b]).wait()

            @pl.when(j + (NBUF + b) < nch)
            def _(b=b):
                pltpu.async_copy(h_ref.at[srcv.at[j + (NBUF + b)]], bufs[b],
                                 gsem[b])


def _sc_agg_body(ha, hb, src_hbm, dst_hbm, outa, outb, srcv, dstv,
                 b0, b1, b2, b3, acc, g0, g1, g2, g3, s0, s1, s2, s3):
    cid = lax.axis_index("c")
    sid = lax.axis_index("s")
    bufs = (b0, b1, b2, b3)
    gsem = (g0, g1, g2, g3)
    ssem = (s0, s1, s2, s3)

    pltpu.sync_copy(src_hbm.at[pl.ds(sid * jnp.int32(CH_FULL), CH_FULL)], srcv)
    pltpu.sync_copy(dst_hbm.at[pl.ds(sid * jnp.int32(CH_FULL), CH_FULL)], dstv)

    _zero_rows(b0, DH)
    @pl.loop(0, RPT // 128)
    def _(r):
        pltpu.sync_copy(b0, acc.at[pl.ds(sid * jnp.int32(RPT) + r * jnp.int32(128), 128)])

    plsc.subcore_barrier()

    @pl.when(cid == 0)
    def _():
        _ring(ha, acc, srcv, dstv, bufs, gsem, ssem, CH_FULL, 128)

    @pl.when(cid == 1)
    def _():
        _ring(hb, acc, srcv, dstv, bufs, gsem, ssem, CH_FULL, 128)

    plsc.subcore_barrier()

    @pl.when(cid == 0)
    def _():
        pltpu.sync_copy(acc.at[pl.ds(sid * jnp.int32(RPT), RPT)],
                        outa.at[pl.ds(sid * jnp.int32(RPT), RPT)])

    @pl.when(cid == 1)
    def _():
        pltpu.sync_copy(acc.at[pl.ds(sid * jnp.int32(RPT), RPT)],
                        outb.at[pl.ds(sid * jnp.int32(RPT), RPT)])


@functools.lru_cache(maxsize=1)
def _sc_agg():
    return pl.kernel(
        _sc_agg_body,
        mesh=_mesh(),
        out_type=[jax.ShapeDtypeStruct((NPAD, DH), f32),
                  jax.ShapeDtypeStruct((NPAD, DH), f32)],
        scratch_types=(
            [pltpu.VMEM((CH_FULL, 128), i32)] * 2
            + [pltpu.VMEM((128, DH), f32)] * NBUF
            + [pltpu.VMEM_SHARED((NPAD, DH), f32)]
            + [pltpu.SemaphoreType.DMA] * (2 * NBUF)
        ),
        compiler_params=pltpu.CompilerParams(use_tc_tiling_on_sc=False),
    )


def _sc_score_body(spre, src_hbm, dst_hbm, outa, outb, srcv, dstv,
                   b0, b1, b2, b3, acc, g0, g1, g2, g3, s0, s1, s2, s3):
    cid = lax.axis_index("c")
    sid = lax.axis_index("s")
    base = cid * jnp.int32(EROWS // 2) + sid * jnp.int32(CH_HALF)
    bufs = (b0, b1, b2, b3)
    gsem = (g0, g1, g2, g3)
    ssem = (s0, s1, s2, s3)

    pltpu.sync_copy(src_hbm.at[pl.ds(base, CH_HALF)], srcv)
    pltpu.sync_copy(dst_hbm.at[pl.ds(base, CH_HALF)], dstv)

    _zero_rows(b0, 16)
    @pl.loop(0, RPT // 128)
    def _(r):
        pltpu.sync_copy(b0, acc.at[pl.ds(sid * jnp.int32(RPT) + r * jnp.int32(128), 128)])

    plsc.subcore_barrier()

    _ring(spre, acc, srcv, dstv, bufs, gsem, ssem, CH_HALF, 128)

    plsc.subcore_barrier()

    @pl.when(cid == 0)
    def _():
        pltpu.sync_copy(acc.at[pl.ds(sid * jnp.int32(RPT), RPT)],
                        outa.at[pl.ds(sid * jnp.int32(RPT), RPT)])

    @pl.when(cid == 1)
    def _():
        pltpu.sync_copy(acc.at[pl.ds(sid * jnp.int32(RPT), RPT)],
                        outb.at[pl.ds(sid * jnp.int32(RPT), RPT)])


@functools.lru_cache(maxsize=1)
def _sc_score():
    return pl.kernel(
        _sc_score_body,
        mesh=_mesh(),
        out_type=[jax.ShapeDtypeStruct((NPAD, 16), f32),
                  jax.ShapeDtypeStruct((NPAD, 16), f32)],
        scratch_types=(
            [pltpu.VMEM((CH_HALF, 128), i32)] * 2
            + [pltpu.VMEM((128, 16), f32)] * NBUF
            + [pltpu.VMEM_SHARED((NPAD, 16), f32)]
            + [pltpu.SemaphoreType.DMA] * (2 * NBUF)
        ),
        compiler_params=pltpu.CompilerParams(use_tc_tiling_on_sc=False),
    )


_HP = dict(preferred_element_type=f32, precision=lax.Precision.HIGHEST)


def _tc_pre(x_ref, dego_ref, degi_ref, h0a_ref, h0b_ref, dout_ref, din_ref):
    dout = lax.rsqrt(jnp.maximum(dego_ref[:, 0:1], 1.0))
    din = lax.rsqrt(jnp.maximum(degi_ref[:, 0:1], 1.0))
    h0 = x_ref[...] * dout
    h0a_ref[...] = h0[:, :DH]
    h0b_ref[...] = h0[:, DH:]
    dout_ref[...] = dout
    din_ref[...] = din


def _layer_core(agga_ref, aggb_ref, din_ref, w_ref, b_ref, g_ref, bb_ref):
    agg = jnp.concatenate([agga_ref[...], aggb_ref[...]], axis=1)
    agg = agg * din_ref[...]
    z = jnp.dot(agg, w_ref[...], **_HP) + b_ref[...]
    mask = (lax.broadcasted_iota(i32, (NPAD, 1), 0) < N).astype(f32)
    m = jnp.sum(z * mask, axis=0, keepdims=True) / N
    ex2 = jnp.sum(z * z * mask, axis=0, keepdims=True) / N
    v = ex2 - m * m
    feat = (z - m) * lax.rsqrt(v + 1e-5) * g_ref[...] + bb_ref[...]
    return jnp.maximum(feat, 0.0)


def _tc_layer(agga_ref, aggb_ref, din_ref, dout_ref, w_ref, b_ref, g_ref,
              bb_ref, ha_ref, hb_ref):
    feat = _layer_core(agga_ref, aggb_ref, din_ref, w_ref, b_ref, g_ref,
                       bb_ref)
    h = feat * dout_ref[...]
    ha_ref[...] = h[:, :DH]
    hb_ref[...] = h[:, DH:]


def _tc_layer3(agga_ref, aggb_ref, din_ref, dout_ref, w_ref, b_ref, g_ref,
               bb_ref, pw_ref, feat_ref, spre_ref):
    feat = _layer_core(agga_ref, aggb_ref, din_ref, w_ref, b_ref, g_ref,
                       bb_ref)
    feat_ref[...] = feat
    h = feat * dout_ref[...]
    spre = jnp.sum(h * pw_ref[...], axis=1, keepdims=True)
    spre_ref[...] = jnp.broadcast_to(spre, (NPAD, 16))


def _tc_top(sca_ref, scb_ref, din_ref, pb_ref, score_ref):
    s = (sca_ref[:, 0:1] + scb_ref[:, 0:1]) * din_ref[...] + pb_ref[0, 0]
    idx = lax.broadcasted_iota(i32, (NPAD, 1), 0)
    score_ref[...] = jnp.where(idx < N, s, -jnp.inf)


def _tc_sel(s_ref, w_ref, m_ref):
    s = s_ref[...]
    b = lax.bitcast_convert_type(s, i32)
    u = jnp.where(b >= 0, b ^ jnp.int32(-2147483648), ~b)

    def vbody(t, carry):
        kk, sel, match = carry
        bit = (31 - t).astype(i32)
        bitv = lax.shift_right_logical(u, bit) & 1
        hi = match * bitv
        c1 = jnp.sum(hi.astype(f32))
        take1 = kk <= c1
        sel = jnp.where(take1, sel, sel | hi)
        kk = jnp.where(take1, kk, kk - c1)
        match = match * jnp.where(take1, bitv, 1 - bitv)
        return kk, sel, match

    init = (jnp.float32(K), jnp.zeros(s.shape, i32),
            jnp.ones(s.shape, i32))
    kk, sel, match = lax.fori_loop(0, 32, vbody, init)

    gidx = (lax.broadcasted_iota(i32, s.shape, 0) * s.shape[1]
            + lax.broadcasted_iota(i32, s.shape, 1))

    def ibody(t, carry):
        kk, sel, match = carry
        bit = (13 - t).astype(i32)
        bitv = lax.shift_right_logical(gidx, bit) & 1
        lo = match * (1 - bitv)
        c0 = jnp.sum(lo.astype(f32))
        take0 = kk <= c0
        sel = jnp.where(take0, sel, sel | lo)
        kk = jnp.where(take0, kk, kk - c0)
        match = match * jnp.where(take0, 1 - bitv, bitv)
        return kk, sel, match

    kk, sel, match = lax.fori_loop(0, 14, ibody, (kk, sel, match))
    sel = sel | match
    mf = sel.astype(f32)
    m_ref[...] = mf
    w_ref[...] = mf * jnp.tanh(s)


def _tc_read(feat_ref, w_ref, m_ref, c1a_ref, c1b_ref, c1bias_ref, c2w_ref,
             c2b_ref, f1w_ref, f1b_ref, f2w_ref, f2b_ref, out_ref):
    feat = feat_ref[...]
    w = w_ref[...]
    m = m_ref[...]
    fw = feat * w
    avg = jnp.sum(fw, axis=0, keepdims=True) / K
    mx = jnp.max(jnp.where(m > 0.5, fw, -jnp.inf), axis=0, keepdims=True)
    h = (jnp.dot(avg, c1a_ref[...], **_HP) + jnp.dot(mx, c1b_ref[...], **_HP)
         + c1bias_ref[...])
    h = jnp.maximum(h, 0.0)
    h = jnp.maximum(jnp.dot(h, c2w_ref[...], **_HP) + c2b_ref[...], 0.0)
    h = jnp.maximum(jnp.dot(h, f1w_ref[...], **_HP) + f1b_ref[...], 0.0)
    o = jnp.dot(h, f2w_ref[...], **_HP) + f2b_ref[...]
    omax = jnp.max(o, axis=1, keepdims=True)
    lse = jnp.log(jnp.sum(jnp.exp(o - omax), axis=1, keepdims=True)) + omax
    out_ref[...] = o - lse


def _sd(shape):
    return jax.ShapeDtypeStruct(shape, f32)


def _call(body, out_shapes, *args):
    return pl.pallas_call(body, out_shape=out_shapes)(*args)


def kernel(*args):
    with jax.enable_x64(False):
        return _kernel32(*args)


def _kernel32(x, edge_index, conv_W0, conv_b0, conv_W1, conv_b1, conv_W2,
              conv_b2, bn_g0, bn_b0, bn_g1, bn_b1, bn_g2, bn_b2, pool_W,
              pool_b, c1_W, c1_b, c2_W, c2_b, fc1_W, fc1_b, fc2_W, fc2_b):
    src = edge_index[0].astype(i32)
    dst = edge_index[1].astype(i32)
    pad_idx = N + (jnp.arange(EPAD - E, dtype=i32) % (NPAD - N))
    src2d = jnp.concatenate([src, pad_idx]).reshape(EROWS, 128)
    dst2d = jnp.concatenate([dst, pad_idx]).reshape(EROWS, 128)
    x_pad = jnp.concatenate([x, jnp.zeros((NPAD - N, 128), f32)], axis=0)

    b0 = conv_b0.reshape(1, 128)
    b1 = conv_b1.reshape(1, 128)
    b2 = conv_b2.reshape(1, 128)
    g0 = bn_g0.reshape(1, 128)
    g1 = bn_g1.reshape(1, 128)
    g2 = bn_g2.reshape(1, 128)
    bb0 = bn_b0.reshape(1, 128)
    bb1 = bn_b1.reshape(1, 128)
    bb2 = bn_b2.reshape(1, 128)
    pw = pool_W.reshape(1, 128)
    pb = pool_b.reshape(1, 1)
    c1a = c1_W[:128]
    c1b = c1_W[128:]
    c1bias = c1_b.reshape(1, 16)
    c2b = c2_b.reshape(1, 32)
    f1b = fc1_b.reshape(1, 128)
    f2b = fc2_b.reshape(1, 2)

    dego, degi = _sc_deg()(src2d, dst2d)
    h0a, h0b, dout, din = _call(
        _tc_pre,
        [_sd((NPAD, DH)), _sd((NPAD, DH)), _sd((NPAD, 1)), _sd((NPAD, 1))],
        x_pad, dego, degi)

    agga, aggb = _sc_agg()(h0a, h0b, src2d, dst2d)
    h1a, h1b = _call(_tc_layer, [_sd((NPAD, DH)), _sd((NPAD, DH))],
                     agga, aggb, din, dout, conv_W0, b0, g0, bb0)

    agga, aggb = _sc_agg()(h1a, h1b, src2d, dst2d)
    h2a, h2b = _call(_tc_layer, [_sd((NPAD, DH)), _sd((NPAD, DH))],
                     agga, aggb, din, dout, conv_W1, b1, g1, bb1)

    agga, aggb = _sc_agg()(h2a, h2b, src2d, dst2d)
    feat3, spre = _call(_tc_layer3, [_sd((NPAD, 128)), _sd((NPAD, 16))],
                        agga, aggb, din, dout, conv_W2, b2, g2, bb2, pw)

    sca, scb = _sc_score()(spre, src2d, dst2d)
    score = _call(_tc_top, _sd((NPAD, 1)), sca, scb, din, pb)

    w80, m80 = _call(_tc_sel, [_sd((NPAD // 128, 128))] * 2,
                     score.reshape(NPAD // 128, 128))

    out = _call(_tc_read, _sd((1, 2)), feat3, w80.reshape(NPAD, 1),
                m80.reshape(NPAD, 1), c1a, c1b, c1bias, c2_W, c2b, fc1_W,
                f1b, fc2_W, f2b)
    return out


# speedup vs baseline: 12.6885x; 1.1452x over previous
import functools

import jax
import jax.numpy as jnp
from jax import lax
from jax.experimental import pallas as pl
from jax.experimental.pallas import tpu as pltpu
from jax.experimental.pallas import tpu_sc as plsc

f32 = jnp.float32
i32 = jnp.int32

N = 10000
NPAD = 10240
E = 320000
EPAD = 327680
EROWS = EPAD // 128
CH_FULL = EROWS // 16
CH_HALF = EROWS // 32
RPT = NPAD // 16
DH = 64
K = 5000

@functools.lru_cache(maxsize=1)
def _mesh():
    return plsc.VectorSubcoreMesh(core_axis_name="c", subcore_axis_name="s")


def _zero_rows(buf, width):
    @pl.loop(0, 128)
    def _(r):
        @pl.loop(0, width, step=16)
        def _(t):
            buf[r, pl.ds(t, 16)] = jnp.zeros((16,), f32)


def _sc_deg_body(src_hbm, dst_hbm, dego_hbm, degi_hbm, idxv, buf, acc, sem):
    cid = lax.axis_index("c")
    sid = lax.axis_index("s")

    @pl.when(cid == 0)
    def _():
        pltpu.sync_copy(src_hbm.at[pl.ds(sid * jnp.int32(CH_FULL), CH_FULL)], idxv)

    @pl.when(cid == 1)
    def _():
        pltpu.sync_copy(dst_hbm.at[pl.ds(sid * jnp.int32(CH_FULL), CH_FULL)], idxv)

    _zero_rows(buf, 16)
    @pl.loop(0, RPT // 128)
    def _(r):
        pltpu.sync_copy(buf, acc.at[pl.ds(sid * jnp.int32(RPT) + r * jnp.int32(128), 128)])

    @pl.loop(0, 128)
    def _(r):
        buf[r, pl.ds(0, 16)] = jnp.full((16,), 1.0, f32)

    plsc.subcore_barrier()

    @pl.loop(0, CH_FULL)
    def _(j):
        pltpu.sync_copy(buf, acc.at[idxv.at[j]], add=True)

    plsc.subcore_barrier()

    @pl.when(cid == 0)
    def _():
        pltpu.sync_copy(acc.at[pl.ds(sid * jnp.int32(RPT), RPT)],
                        dego_hbm.at[pl.ds(sid * jnp.int32(RPT), RPT)])

    @pl.when(cid == 1)
    def _():
        pltpu.sync_copy(acc.at[pl.ds(sid * jnp.int32(RPT), RPT)],
                        degi_hbm.at[pl.ds(sid * jnp.int32(RPT), RPT)])


@functools.lru_cache(maxsize=1)
def _sc_deg():
    return pl.kernel(
        _sc_deg_body,
        mesh=_mesh(),
        out_type=[jax.ShapeDtypeStruct((NPAD, 16), f32),
                  jax.ShapeDtypeStruct((NPAD, 16), f32)],
        scratch_types=[
            pltpu.VMEM((CH_FULL, 128), i32),
            pltpu.VMEM((128, 16), f32),
            pltpu.VMEM_SHARED((NPAD, 16), f32),
            pltpu.SemaphoreType.DMA,
        ],
        compiler_params=pltpu.CompilerParams(use_tc_tiling_on_sc=False),
    )


NBUF = 4


def _ring(h_ref, acc, srcv, dstv, bufs, gsem, ssem, nch, rows):
    for b in range(NBUF):
        pltpu.async_copy(h_ref.at[srcv.at[b]], bufs[b], gsem[b])

    @pl.loop(0, nch, step=NBUF)
    def _(j):
        for b in range(NBUF):
            pltpu.make_async_copy(h_ref.at[pl.ds(0, rows)], bufs[b],
                                  gsem[b]).wait()
            pltpu.async_copy(bufs[b], acc.at[dstv.at[j + b]], ssem[b],
                             add=True)
        for b in range(NBUF):
            pltpu.make_async_copy(bufs[b], acc.at[pl.ds(0, rows)],
                                  ssem[b]).wait()

            @pl.when(j + (NBUF + b) < nch)
            def _(b=b):
                pltpu.async_copy(h_ref.at[srcv.at[j + (NBUF + b)]], bufs[b],
                                 gsem[b])


def _sc_agg_body(ha, hb, src_hbm, dst_hbm, outa, outb, srcv, dstv,
                 b0, b1, b2, b3, acc, g0, g1, g2, g3, s0, s1, s2, s3):
    cid = lax.axis_index("c")
    sid = lax.axis_index("s")
    bufs = (b0, b1, b2, b3)
    gsem = (g0, g1, g2, g3)
    ssem = (s0, s1, s2, s3)

    pltpu.sync_copy(src_hbm.at[pl.ds(sid * jnp.int32(CH_FULL), CH_FULL)], srcv)
    pltpu.sync_copy(dst_hbm.at[pl.ds(sid * jnp.int32(CH_FULL), CH_FULL)], dstv)

    _zero_rows(b0, DH)
    @pl.loop(0, RPT // 128)
    def _(r):
        pltpu.sync_copy(b0, acc.at[pl.ds(sid * jnp.int32(RPT) + r * jnp.int32(128), 128)])

    plsc.subcore_barrier()

    @pl.when(cid == 0)
    def _():
        _ring(ha, acc, srcv, dstv, bufs, gsem, ssem, CH_FULL, 128)

    @pl.when(cid == 1)
    def _():
        _ring(hb, acc, srcv, dstv, bufs, gsem, ssem, CH_FULL, 128)

    plsc.subcore_barrier()

    @pl.when(cid == 0)
    def _():
        pltpu.sync_copy(acc.at[pl.ds(sid * jnp.int32(RPT), RPT)],
                        outa.at[pl.ds(sid * jnp.int32(RPT), RPT)])

    @pl.when(cid == 1)
    def _():
        pltpu.sync_copy(acc.at[pl.ds(sid * jnp.int32(RPT), RPT)],
                        outb.at[pl.ds(sid * jnp.int32(RPT), RPT)])


@functools.lru_cache(maxsize=1)
def _sc_agg():
    return pl.kernel(
        _sc_agg_body,
        mesh=_mesh(),
        out_type=[jax.ShapeDtypeStruct((NPAD, DH), f32),
                  jax.ShapeDtypeStruct((NPAD, DH), f32)],
        scratch_types=(
            [pltpu.VMEM((CH_FULL, 128), i32)] * 2
            + [pltpu.VMEM((128, DH), f32)] * NBUF
            + [pltpu.VMEM_SHARED((NPAD, DH), f32)]
            + [pltpu.SemaphoreType.DMA] * (2 * NBUF)
        ),
        compiler_params=pltpu.CompilerParams(use_tc_tiling_on_sc=False),
    )


def _sc_score_body(spre, src_hbm, dst_hbm, outa, outb, srcv, dstv,
                   b0, b1, b2, b3, acc, g0, g1, g2, g3, s0, s1, s2, s3):
    cid = lax.axis_index("c")
    sid = lax.axis_index("s")
    base = cid * jnp.int32(EROWS // 2) + sid * jnp.int32(CH_HALF)
    bufs = (b0, b1, b2, b3)
    gsem = (g0, g1, g2, g3)
    ssem = (s0, s1, s2, s3)

    pltpu.sync_copy(src_hbm.at[pl.ds(base, CH_HALF)], srcv)
    pltpu.sync_copy(dst_hbm.at[pl.ds(base, CH_HALF)], dstv)

    _zero_rows(b0, 16)
    @pl.loop(0, RPT // 128)
    def _(r):
        pltpu.sync_copy(b0, acc.at[pl.ds(sid * jnp.int32(RPT) + r * jnp.int32(128), 128)])

    plsc.subcore_barrier()

    _ring(spre, acc, srcv, dstv, bufs, gsem, ssem, CH_HALF, 128)

    plsc.subcore_barrier()

    @pl.when(cid == 0)
    def _():
        pltpu.sync_copy(acc.at[pl.ds(sid * jnp.int32(RPT), RPT)],
                        outa.at[pl.ds(sid * jnp.int32(RPT), RPT)])

    @pl.when(cid == 1)
    def _():
        pltpu.sync_copy(acc.at[pl.ds(sid * jnp.int32(RPT), RPT)],
                        outb.at[pl.ds(sid * jnp.int32(RPT), RPT)])


@functools.lru_cache(maxsize=1)
def _sc_score():
    return pl.kernel(
        _sc_score_body,
        mesh=_mesh(),
        out_type=[jax.ShapeDtypeStruct((NPAD, 16), f32),
                  jax.ShapeDtypeStruct((NPAD, 16), f32)],
        scratch_types=(
            [pltpu.VMEM((CH_HALF, 128), i32)] * 2
            + [pltpu.VMEM((128, 16), f32)] * NBUF
            + [pltpu.VMEM_SHARED((NPAD, 16), f32)]
            + [pltpu.SemaphoreType.DMA] * (2 * NBUF)
        ),
        compiler_params=pltpu.CompilerParams(use_tc_tiling_on_sc=False),
    )


_HP = dict(preferred_element_type=f32)


NH = NPAD // 2
NMASK = N // 2


def _tc_pre(x_ref, dego_ref, degi_ref, ha_ref, hb_ref, da_ref, db_ref,
            ia_ref, ib_ref):
    douta = lax.rsqrt(jnp.maximum(dego_ref[:, 0:1], 1.0))
    doutb = lax.rsqrt(jnp.maximum(dego_ref[:, 16:17], 1.0))
    dina = lax.rsqrt(jnp.maximum(degi_ref[:, 0:1], 1.0))
    dinb = lax.rsqrt(jnp.maximum(degi_ref[:, 16:17], 1.0))
    x5 = x_ref[...]
    he = x5[:, :128] * douta
    ho = x5[:, 128:] * doutb
    ha_ref[...] = jnp.concatenate([he[:, :DH], ho[:, :DH]], axis=1)
    hb_ref[...] = jnp.concatenate([he[:, DH:], ho[:, DH:]], axis=1)
    da_ref[...] = douta
    db_ref[...] = doutb
    ia_ref[...] = dina
    ib_ref[...] = dinb


def _layer_core(agga_ref, aggb_ref, ia_ref, ib_ref, w_ref, b_ref, g_ref,
                bb_ref):
    agga = agga_ref[...]
    aggb = aggb_ref[...]
    even = jnp.concatenate([agga[:, :DH], aggb[:, :DH]], axis=1) * ia_ref[...]
    odd = jnp.concatenate([agga[:, DH:], aggb[:, DH:]], axis=1) * ib_ref[...]
    ze = jnp.dot(even, w_ref[...], **_HP) + b_ref[...]
    zo = jnp.dot(odd, w_ref[...], **_HP) + b_ref[...]
    mask = (lax.broadcasted_iota(i32, (NH, 1), 0) < NMASK).astype(f32)
    zem = ze * mask
    zom = zo * mask
    m = (jnp.sum(zem, axis=0, keepdims=True)
         + jnp.sum(zom, axis=0, keepdims=True)) / N
    ex2 = (jnp.sum(ze * zem, axis=0, keepdims=True)
           + jnp.sum(zo * zom, axis=0, keepdims=True)) / N
    sc = lax.rsqrt(ex2 - m * m + 1e-5)
    fe = jnp.maximum((ze - m) * sc * g_ref[...] + bb_ref[...], 0.0)
    fo = jnp.maximum((zo - m) * sc * g_ref[...] + bb_ref[...], 0.0)
    return fe, fo


def _tc_layer(agga_ref, aggb_ref, ia_ref, ib_ref, da_ref, db_ref, w_ref,
              b_ref, g_ref, bb_ref, ha_ref, hb_ref):
    fe, fo = _layer_core(agga_ref, aggb_ref, ia_ref, ib_ref, w_ref, b_ref,
                         g_ref, bb_ref)
    he = fe * da_ref[...]
    ho = fo * db_ref[...]
    ha_ref[...] = jnp.concatenate([he[:, :DH], ho[:, :DH]], axis=1)
    hb_ref[...] = jnp.concatenate([he[:, DH:], ho[:, DH:]], axis=1)


def _tc_layer3(agga_ref, aggb_ref, ia_ref, ib_ref, da_ref, db_ref, w_ref,
               b_ref, g_ref, bb_ref, pw_ref, fe_ref, fo_ref, spre_ref):
    fe, fo = _layer_core(agga_ref, aggb_ref, ia_ref, ib_ref, w_ref, b_ref,
                         g_ref, bb_ref)
    fe_ref[...] = fe
    fo_ref[...] = fo
    pw = pw_ref[...]
    spre_e = jnp.sum(fe * da_ref[...] * pw, axis=1, keepdims=True)
    spre_o = jnp.sum(fo * db_ref[...] * pw, axis=1, keepdims=True)
    spre_ref[...] = jnp.concatenate(
        [jnp.broadcast_to(spre_e, (NH, 16)),
         jnp.broadcast_to(spre_o, (NH, 16))], axis=1)


def _tc_top(sca_ref, scb_ref, ia_ref, ib_ref, pb_ref, score_ref):
    pb = pb_ref[0, 0]
    se = (sca_ref[:, 0:1] + scb_ref[:, 0:1]) * ia_ref[...] + pb
    so = (sca_ref[:, 16:17] + scb_ref[:, 16:17]) * ib_ref[...] + pb
    real = lax.broadcasted_iota(i32, (NH, 1), 0) < NMASK
    se = jnp.where(real, se, -jnp.inf)
    so = jnp.where(real, so, -jnp.inf)
    score_ref[...] = jnp.concatenate([se, so], axis=1)


def _tc_sel(s_ref, w_ref, m_ref):
    s = s_ref[...]
    b = lax.bitcast_convert_type(s, i32)
    u = jnp.where(b >= 0, b ^ jnp.int32(-2147483648), ~b)

    def vbody(t, carry):
        kk, sel, match = carry
        bit = (31 - t).astype(i32)
        bitv = lax.shift_right_logical(u, bit) & 1
        hi = match * bitv
        c1 = jnp.sum(hi.astype(f32))
        take1 = kk <= c1
        sel = jnp.where(take1, sel, sel | hi)
        kk = jnp.where(take1, kk, kk - c1)
        match = match * jnp.where(take1, bitv, 1 - bitv)
        return kk, sel, match

    init = (jnp.float32(K), jnp.zeros(s.shape, i32),
            jnp.ones(s.shape, i32))
    kk, sel, match = lax.fori_loop(0, 32, vbody, init)

    gidx = (lax.broadcasted_iota(i32, s.shape, 0) * s.shape[1]
            + lax.broadcasted_iota(i32, s.shape, 1))

    def ibody(t, carry):
        kk, sel, match = carry
        bit = (13 - t).astype(i32)
        bitv = lax.shift_right_logical(gidx, bit) & 1
        lo = match * (1 - bitv)
        c0 = jnp.sum(lo.astype(f32))
        take0 = kk <= c0
        sel = jnp.where(take0, sel, sel | lo)
        kk = jnp.where(take0, kk, kk - c0)
        match = match * jnp.where(take0, 1 - bitv, bitv)
        return kk, sel, match

    kk, sel, match = lax.fori_loop(0, 14, ibody, (kk, sel, match))
    sel = sel | match
    mf = sel.astype(f32)
    m_ref[...] = mf
    w_ref[...] = mf * jnp.tanh(s)


def _tc_read(fe_ref, fo_ref, wm_ref, c1a_ref, c1b_ref, c1bias_ref, c2w_ref,
             c2b_ref, f1w_ref, f1b_ref, f2w_ref, f2b_ref, out_ref):
    wm = wm_ref[...]
    fwe = fe_ref[...] * wm[:, 0:1]
    fwo = fo_ref[...] * wm[:, 1:2]
    avg = (jnp.sum(fwe, axis=0, keepdims=True)
           + jnp.sum(fwo, axis=0, keepdims=True)) / K
    mx = jnp.maximum(
        jnp.max(jnp.where(wm[:, 2:3] > 0.5, fwe, -jnp.inf), axis=0,
                keepdims=True),
        jnp.max(jnp.where(wm[:, 3:4] > 0.5, fwo, -jnp.inf), axis=0,
                keepdims=True))
    h = (jnp.dot(avg, c1a_ref[...], **_HP) + jnp.dot(mx, c1b_ref[...], **_HP)
         + c1bias_ref[...])
    h = jnp.maximum(h, 0.0)
    h = jnp.maximum(jnp.dot(h, c2w_ref[...], **_HP) + c2b_ref[...], 0.0)
    h = jnp.maximum(jnp.dot(h, f1w_ref[...], **_HP) + f1b_ref[...], 0.0)
    o = jnp.dot(h, f2w_ref[...], **_HP) + f2b_ref[...]
    omax = jnp.max(o, axis=1, keepdims=True)
    lse = jnp.log(jnp.sum(jnp.exp(o - omax), axis=1, keepdims=True)) + omax
    out_ref[...] = o - lse


def _sd(shape):
    return jax.ShapeDtypeStruct(shape, f32)


def _call(body, out_shapes, *args):
    return pl.pallas_call(body, out_shape=out_shapes)(*args)


def kernel(*args):
    with jax.enable_x64(False):
        return _kernel32(*args)


def _kernel32(x, edge_index, conv_W0, conv_b0, conv_W1, conv_b1, conv_W2,
              conv_b2, bn_g0, bn_b0, bn_g1, bn_b1, bn_g2, bn_b2, pool_W,
              pool_b, c1_W, c1_b, c2_W, c2_b, fc1_W, fc1_b, fc2_W, fc2_b):
    src = edge_index[0].astype(i32)
    dst = edge_index[1].astype(i32)
    pad_idx = N + (jnp.arange(EPAD - E, dtype=i32) % (NPAD - N))
    src2d = jnp.concatenate([src, pad_idx]).reshape(EROWS, 128)
    dst2d = jnp.concatenate([dst, pad_idx]).reshape(EROWS, 128)
    x_pad = jnp.concatenate([x, jnp.zeros((NPAD - N, 128), f32)], axis=0)

    b0 = conv_b0.reshape(1, 128)
    b1 = conv_b1.reshape(1, 128)
    b2 = conv_b2.reshape(1, 128)
    g0 = bn_g0.reshape(1, 128)
    g1 = bn_g1.reshape(1, 128)
    g2 = bn_g2.reshape(1, 128)
    bb0 = bn_b0.reshape(1, 128)
    bb1 = bn_b1.reshape(1, 128)
    bb2 = bn_b2.reshape(1, 128)
    pw = pool_W.reshape(1, 128)
    pb = pool_b.reshape(1, 1)
    c1a = c1_W[:128]
    c1b = c1_W[128:]
    c1bias = c1_b.reshape(1, 16)
    c2b = c2_b.reshape(1, 32)
    f1b = fc1_b.reshape(1, 128)
    f2b = fc2_b.reshape(1, 2)

    dego, degi = _sc_deg()(src2d, dst2d)
    h0a5, h0b5, da, db, ia, ib = _call(
        _tc_pre,
        [_sd((NH, 128)), _sd((NH, 128))] + [_sd((NH, 1))] * 4,
        x_pad.reshape(NH, 256), dego.reshape(NH, 32), degi.reshape(NH, 32))

    agga, aggb = _sc_agg()(h0a5.reshape(NPAD, DH), h0b5.reshape(NPAD, DH),
                           src2d, dst2d)
    h1a5, h1b5 = _call(_tc_layer, [_sd((NH, 128))] * 2,
                       agga.reshape(NH, 128), aggb.reshape(NH, 128),
                       ia, ib, da, db, conv_W0, b0, g0, bb0)

    agga, aggb = _sc_agg()(h1a5.reshape(NPAD, DH), h1b5.reshape(NPAD, DH),
                           src2d, dst2d)
    h2a5, h2b5 = _call(_tc_layer, [_sd((NH, 128))] * 2,
                       agga.reshape(NH, 128), aggb.reshape(NH, 128),
                       ia, ib, da, db, conv_W1, b1, g1, bb1)

    agga, aggb = _sc_agg()(h2a5.reshape(NPAD, DH), h2b5.reshape(NPAD, DH),
                           src2d, dst2d)
    fe, fo, spre = _call(
        _tc_layer3, [_sd((NH, 128)), _sd((NH, 128)), _sd((NH, 32))],
        agga.reshape(NH, 128), aggb.reshape(NH, 128),
        ia, ib, da, db, conv_W2, b2, g2, bb2, pw)

    sca, scb = _sc_score()(spre.reshape(NPAD, 16), src2d, dst2d)
    score = _call(_tc_top, _sd((NH, 2)), sca.reshape(NH, 32),
                  scb.reshape(NH, 32), ia, ib, pb)

    w80, m80 = _call(_tc_sel, [_sd((NPAD // 128, 128))] * 2,
                     score.reshape(NPAD // 128, 128))

    wm = jnp.concatenate([w80.reshape(NH, 2), m80.reshape(NH, 2)], axis=1)
    out = _call(_tc_read, _sd((1, 2)), fe, fo, wm, c1a, c1b, c1bias, c2_W,
                c2b, fc1_W, f1b, fc2_W, f2b)
    return out


# speedup vs baseline: 12.7331x; 1.0035x over previous
import functools

import jax
import jax.numpy as jnp
from jax import lax
from jax.experimental import pallas as pl
from jax.experimental.pallas import tpu as pltpu
from jax.experimental.pallas import tpu_sc as plsc

f32 = jnp.float32
i32 = jnp.int32

N = 10000
NPAD = 10240
E = 320000
EPAD = 327680
EROWS = EPAD // 128
CH_FULL = EROWS // 16
CH_HALF = EROWS // 32
RPT = NPAD // 16
DH = 64
K = 5000

@functools.lru_cache(maxsize=1)
def _mesh():
    return plsc.VectorSubcoreMesh(core_axis_name="c", subcore_axis_name="s")


def _zero_rows(buf, width):
    @pl.loop(0, 128)
    def _(r):
        @pl.loop(0, width, step=16)
        def _(t):
            buf[r, pl.ds(t, 16)] = jnp.zeros((16,), f32)


def _sc_deg_body(src_hbm, dst_hbm, o0_hbm, o1_hbm, i0_hbm, i1_hbm,
                 sidxv, didxv, buf, acco, acci, sem):
    cid = lax.axis_index("c")
    sid = lax.axis_index("s")
    base = cid * jnp.int32(EROWS // 2) + sid * jnp.int32(CH_HALF)

    pltpu.sync_copy(src_hbm.at[pl.ds(base, CH_HALF)], sidxv)
    pltpu.sync_copy(dst_hbm.at[pl.ds(base, CH_HALF)], didxv)

    _zero_rows(buf, 16)
    @pl.loop(0, RPT // 128)
    def _(r):
        pltpu.sync_copy(buf, acco.at[pl.ds(sid * jnp.int32(RPT) + r * jnp.int32(128), 128)])
        pltpu.sync_copy(buf, acci.at[pl.ds(sid * jnp.int32(RPT) + r * jnp.int32(128), 128)])

    @pl.loop(0, 128)
    def _(r):
        buf[r, pl.ds(0, 16)] = jnp.full((16,), 1.0, f32)

    plsc.subcore_barrier()

    @pl.loop(0, CH_HALF)
    def _(j):
        pltpu.sync_copy(buf, acco.at[sidxv.at[j]], add=True)
        pltpu.sync_copy(buf, acci.at[didxv.at[j]], add=True)

    plsc.subcore_barrier()

    @pl.when(cid == 0)
    def _():
        pltpu.sync_copy(acco.at[pl.ds(sid * jnp.int32(RPT), RPT)],
                        o0_hbm.at[pl.ds(sid * jnp.int32(RPT), RPT)])
        pltpu.sync_copy(acci.at[pl.ds(sid * jnp.int32(RPT), RPT)],
                        i0_hbm.at[pl.ds(sid * jnp.int32(RPT), RPT)])

    @pl.when(cid == 1)
    def _():
        pltpu.sync_copy(acco.at[pl.ds(sid * jnp.int32(RPT), RPT)],
                        o1_hbm.at[pl.ds(sid * jnp.int32(RPT), RPT)])
        pltpu.sync_copy(acci.at[pl.ds(sid * jnp.int32(RPT), RPT)],
                        i1_hbm.at[pl.ds(sid * jnp.int32(RPT), RPT)])


@functools.lru_cache(maxsize=1)
def _sc_deg():
    return pl.kernel(
        _sc_deg_body,
        mesh=_mesh(),
        out_type=[jax.ShapeDtypeStruct((NPAD, 16), f32)] * 4,
        scratch_types=[
            pltpu.VMEM((CH_HALF, 128), i32),
            pltpu.VMEM((CH_HALF, 128), i32),
            pltpu.VMEM((128, 16), f32),
            pltpu.VMEM_SHARED((NPAD, 16), f32),
            pltpu.VMEM_SHARED((NPAD, 16), f32),
            pltpu.SemaphoreType.DMA,
        ],
        compiler_params=pltpu.CompilerParams(use_tc_tiling_on_sc=False),
    )


NBUF = 5


def _ring(h_ref, acc, srcv, dstv, bufs, gsem, ssem, nch, rows):
    for b in range(NBUF):
        pltpu.async_copy(h_ref.at[srcv.at[b]], bufs[b], gsem[b])

    @pl.loop(0, nch, step=NBUF)
    def _(j):
        for b in range(NBUF):
            pltpu.make_async_copy(h_ref.at[pl.ds(0, rows)], bufs[b],
                                  gsem[b]).wait()
            pltpu.async_copy(bufs[b], acc.at[dstv.at[j + b]], ssem[b],
                             add=True)
        for b in range(NBUF):
            pltpu.make_async_copy(bufs[b], acc.at[pl.ds(0, rows)],
                                  ssem[b]).wait()

            @pl.when(j + (NBUF + b) < nch)
            def _(b=b):
                pltpu.async_copy(h_ref.at[srcv.at[j + (NBUF + b)]], bufs[b],
                                 gsem[b])


def _sc_agg_body(ha, hb, src_hbm, dst_hbm, outa, outb, srcv, dstv,
                 b0, b1, b2, b3, b4, acc,
                 g0, g1, g2, g3, g4, s0, s1, s2, s3, s4):
    cid = lax.axis_index("c")
    sid = lax.axis_index("s")
    bufs = (b0, b1, b2, b3, b4)
    gsem = (g0, g1, g2, g3, g4)
    ssem = (s0, s1, s2, s3, s4)

    pltpu.sync_copy(src_hbm.at[pl.ds(sid * jnp.int32(CH_FULL), CH_FULL)], srcv)
    pltpu.sync_copy(dst_hbm.at[pl.ds(sid * jnp.int32(CH_FULL), CH_FULL)], dstv)

    _zero_rows(b0, DH)
    @pl.loop(0, RPT // 128)
    def _(r):
        pltpu.sync_copy(b0, acc.at[pl.ds(sid * jnp.int32(RPT) + r * jnp.int32(128), 128)])

    plsc.subcore_barrier()

    @pl.when(cid == 0)
    def _():
        _ring(ha, acc, srcv, dstv, bufs, gsem, ssem, CH_FULL, 128)

    @pl.when(cid == 1)
    def _():
        _ring(hb, acc, srcv, dstv, bufs, gsem, ssem, CH_FULL, 128)

    plsc.subcore_barrier()

    @pl.when(cid == 0)
    def _():
        pltpu.sync_copy(acc.at[pl.ds(sid * jnp.int32(RPT), RPT)],
                        outa.at[pl.ds(sid * jnp.int32(RPT), RPT)])

    @pl.when(cid == 1)
    def _():
        pltpu.sync_copy(acc.at[pl.ds(sid * jnp.int32(RPT), RPT)],
                        outb.at[pl.ds(sid * jnp.int32(RPT), RPT)])


@functools.lru_cache(maxsize=1)
def _sc_agg():
    return pl.kernel(
        _sc_agg_body,
        mesh=_mesh(),
        out_type=[jax.ShapeDtypeStruct((NPAD, DH), f32),
                  jax.ShapeDtypeStruct((NPAD, DH), f32)],
        scratch_types=(
            [pltpu.VMEM((CH_FULL, 128), i32)] * 2
            + [pltpu.VMEM((128, DH), f32)] * NBUF
            + [pltpu.VMEM_SHARED((NPAD, DH), f32)]
            + [pltpu.SemaphoreType.DMA] * (2 * NBUF)
        ),
        compiler_params=pltpu.CompilerParams(use_tc_tiling_on_sc=False),
    )


def _sc_score_body(spre, src_hbm, dst_hbm, outa, outb, srcv, dstv,
                   b0, b1, b2, b3, b4, acc,
                   g0, g1, g2, g3, g4, s0, s1, s2, s3, s4):
    cid = lax.axis_index("c")
    sid = lax.axis_index("s")
    base = cid * jnp.int32(EROWS // 2) + sid * jnp.int32(CH_HALF)
    bufs = (b0, b1, b2, b3, b4)
    gsem = (g0, g1, g2, g3, g4)
    ssem = (s0, s1, s2, s3, s4)

    pltpu.sync_copy(src_hbm.at[pl.ds(base, CH_HALF)], srcv)
    pltpu.sync_copy(dst_hbm.at[pl.ds(base, CH_HALF)], dstv)

    _zero_rows(b0, 16)
    @pl.loop(0, RPT // 128)
    def _(r):
        pltpu.sync_copy(b0, acc.at[pl.ds(sid * jnp.int32(RPT) + r * jnp.int32(128), 128)])

    plsc.subcore_barrier()

    _ring(spre, acc, srcv, dstv, bufs, gsem, ssem, CH_HALF, 128)

    plsc.subcore_barrier()

    @pl.when(cid == 0)
    def _():
        pltpu.sync_copy(acc.at[pl.ds(sid * jnp.int32(RPT), RPT)],
                        outa.at[pl.ds(sid * jnp.int32(RPT), RPT)])

    @pl.when(cid == 1)
    def _():
        pltpu.sync_copy(acc.at[pl.ds(sid * jnp.int32(RPT), RPT)],
                        outb.at[pl.ds(sid * jnp.int32(RPT), RPT)])


@functools.lru_cache(maxsize=1)
def _sc_score():
    return pl.kernel(
        _sc_score_body,
        mesh=_mesh(),
        out_type=[jax.ShapeDtypeStruct((NPAD, 16), f32),
                  jax.ShapeDtypeStruct((NPAD, 16), f32)],
        scratch_types=(
            [pltpu.VMEM((CH_HALF, 128), i32)] * 2
            + [pltpu.VMEM((128, 16), f32)] * NBUF
            + [pltpu.VMEM_SHARED((NPAD, 16), f32)]
            + [pltpu.SemaphoreType.DMA] * (2 * NBUF)
        ),
        compiler_params=pltpu.CompilerParams(use_tc_tiling_on_sc=False),
    )


_HP = dict(preferred_element_type=f32)


NH = NPAD // 2
NMASK = N // 2


def _tc_pre(x_ref, o0_ref, o1_ref, i0_ref, i1_ref, ha_ref, hb_ref,
            da_ref, db_ref, ia_ref, ib_ref):
    dego = o0_ref[...] + o1_ref[...]
    degi = i0_ref[...] + i1_ref[...]
    douta = lax.rsqrt(jnp.maximum(dego[:, 0:1], 1.0))
    doutb = lax.rsqrt(jnp.maximum(dego[:, 16:17], 1.0))
    dina = lax.rsqrt(jnp.maximum(degi[:, 0:1], 1.0))
    dinb = lax.rsqrt(jnp.maximum(degi[:, 16:17], 1.0))
    x5 = x_ref[...]
    he = x5[:, :128] * douta
    ho = x5[:, 128:] * doutb
    ha_ref[...] = jnp.concatenate([he[:, :DH], ho[:, :DH]], axis=1)
    hb_ref[...] = jnp.concatenate([he[:, DH:], ho[:, DH:]], axis=1)
    da_ref[...] = douta
    db_ref[...] = doutb
    ia_ref[...] = dina
    ib_ref[...] = dinb


def _layer_core(agga_ref, aggb_ref, ia_ref, ib_ref, w_ref, b_ref, g_ref,
                bb_ref):
    agga = agga_ref[...]
    aggb = aggb_ref[...]
    even = jnp.concatenate([agga[:, :DH], aggb[:, :DH]], axis=1) * ia_ref[...]
    odd = jnp.concatenate([agga[:, DH:], aggb[:, DH:]], axis=1) * ib_ref[...]
    ze = jnp.dot(even, w_ref[...], **_HP) + b_ref[...]
    zo = jnp.dot(odd, w_ref[...], **_HP) + b_ref[...]
    mask = (lax.broadcasted_iota(i32, (NH, 1), 0) < NMASK).astype(f32)
    zem = ze * mask
    zom = zo * mask
    m = (jnp.sum(zem, axis=0, keepdims=True)
         + jnp.sum(zom, axis=0, keepdims=True)) / N
    ex2 = (jnp.sum(ze * zem, axis=0, keepdims=True)
           + jnp.sum(zo * zom, axis=0, keepdims=True)) / N
    sc = lax.rsqrt(ex2 - m * m + 1e-5)
    fe = jnp.maximum((ze - m) * sc * g_ref[...] + bb_ref[...], 0.0)
    fo = jnp.maximum((zo - m) * sc * g_ref[...] + bb_ref[...], 0.0)
    return fe, fo


def _tc_layer(agga_ref, aggb_ref, ia_ref, ib_ref, da_ref, db_ref, w_ref,
              b_ref, g_ref, bb_ref, ha_ref, hb_ref):
    fe, fo = _layer_core(agga_ref, aggb_ref, ia_ref, ib_ref, w_ref, b_ref,
                         g_ref, bb_ref)
    he = fe * da_ref[...]
    ho = fo * db_ref[...]
    ha_ref[...] = jnp.concatenate([he[:, :DH], ho[:, :DH]], axis=1)
    hb_ref[...] = jnp.concatenate([he[:, DH:], ho[:, DH:]], axis=1)


def _tc_layer3(agga_ref, aggb_ref, ia_ref, ib_ref, da_ref, db_ref, w_ref,
               b_ref, g_ref, bb_ref, pw_ref, fe_ref, fo_ref, spre_ref):
    fe, fo = _layer_core(agga_ref, aggb_ref, ia_ref, ib_ref, w_ref, b_ref,
                         g_ref, bb_ref)
    fe_ref[...] = fe
    fo_ref[...] = fo
    pw = pw_ref[...]
    spre_e = jnp.sum(fe * da_ref[...] * pw, axis=1, keepdims=True)
    spre_o = jnp.sum(fo * db_ref[...] * pw, axis=1, keepdims=True)
    spre_ref[...] = jnp.concatenate(
        [jnp.broadcast_to(spre_e, (NH, 16)),
         jnp.broadcast_to(spre_o, (NH, 16))], axis=1)


def _tc_top(sca_ref, scb_ref, ia_ref, ib_ref, pb_ref, score_ref):
    pb = pb_ref[0, 0]
    se = (sca_ref[:, 0:1] + scb_ref[:, 0:1]) * ia_ref[...] + pb
    so = (sca_ref[:, 16:17] + scb_ref[:, 16:17]) * ib_ref[...] + pb
    real = lax.broadcasted_iota(i32, (NH, 1), 0) < NMASK
    se = jnp.where(real, se, -jnp.inf)
    so = jnp.where(real, so, -jnp.inf)
    score_ref[...] = jnp.concatenate([se, so], axis=1)


def _tc_sel(s_ref, w_ref, m_ref):
    s = s_ref[...]
    b = lax.bitcast_convert_type(s, i32)
    u = jnp.where(b >= 0, b ^ jnp.int32(-2147483648), ~b)

    def vbody(t, carry):
        kk, sel, match = carry
        bit = (31 - t).astype(i32)
        bitv = lax.shift_right_logical(u, bit) & 1
        hi = match * bitv
        c1 = jnp.sum(hi.astype(f32))
        take1 = kk <= c1
        sel = jnp.where(take1, sel, sel | hi)
        kk = jnp.where(take1, kk, kk - c1)
        match = match * jnp.where(take1, bitv, 1 - bitv)
        return kk, sel, match

    init = (jnp.float32(K), jnp.zeros(s.shape, i32),
            jnp.ones(s.shape, i32))
    kk, sel, match = lax.fori_loop(0, 32, vbody, init)

    gidx = (lax.broadcasted_iota(i32, s.shape, 0) * s.shape[1]
            + lax.broadcasted_iota(i32, s.shape, 1))

    def ibody(t, carry):
        kk, sel, match = carry
        bit = (13 - t).astype(i32)
        bitv = lax.shift_right_logical(gidx, bit) & 1
        lo = match * (1 - bitv)
        c0 = jnp.sum(lo.astype(f32))
        take0 = kk <= c0
        sel = jnp.where(take0, sel, sel | lo)
        kk = jnp.where(take0, kk, kk - c0)
        match = match * jnp.where(take0, 1 - bitv, bitv)
        return kk, sel, match

    kk, sel, match = lax.fori_loop(0, 14, ibody, (kk, sel, match))
    sel = sel | match
    mf = sel.astype(f32)
    m_ref[...] = mf
    w_ref[...] = mf * jnp.tanh(s)


def _tc_read(fe_ref, fo_ref, wm_ref, c1a_ref, c1b_ref, c1bias_ref, c2w_ref,
             c2b_ref, f1w_ref, f1b_ref, f2w_ref, f2b_ref, out_ref):
    wm = wm_ref[...]
    fwe = fe_ref[...] * wm[:, 0:1]
    fwo = fo_ref[...] * wm[:, 1:2]
    avg = (jnp.sum(fwe, axis=0, keepdims=True)
           + jnp.sum(fwo, axis=0, keepdims=True)) / K
    mx = jnp.maximum(
        jnp.max(jnp.where(wm[:, 2:3] > 0.5, fwe, -jnp.inf), axis=0,
                keepdims=True),
        jnp.max(jnp.where(wm[:, 3:4] > 0.5, fwo, -jnp.inf), axis=0,
                keepdims=True))
    h = (jnp.dot(avg, c1a_ref[...], **_HP) + jnp.dot(mx, c1b_ref[...], **_HP)
         + c1bias_ref[...])
    h = jnp.maximum(h, 0.0)
    h = jnp.maximum(jnp.dot(h, c2w_ref[...], **_HP) + c2b_ref[...], 0.0)
    h = jnp.maximum(jnp.dot(h, f1w_ref[...], **_HP) + f1b_ref[...], 0.0)
    o = jnp.dot(h, f2w_ref[...], **_HP) + f2b_ref[...]
    omax = jnp.max(o, axis=1, keepdims=True)
    lse = jnp.log(jnp.sum(jnp.exp(o - omax), axis=1, keepdims=True)) + omax
    out_ref[...] = o - lse


def _sd(shape):
    return jax.ShapeDtypeStruct(shape, f32)


def _call(body, out_shapes, *args):
    return pl.pallas_call(body, out_shape=out_shapes)(*args)


def kernel(*args):
    with jax.enable_x64(False):
        return _kernel32(*args)


def _kernel32(x, edge_index, conv_W0, conv_b0, conv_W1, conv_b1, conv_W2,
              conv_b2, bn_g0, bn_b0, bn_g1, bn_b1, bn_g2, bn_b2, pool_W,
              pool_b, c1_W, c1_b, c2_W, c2_b, fc1_W, fc1_b, fc2_W, fc2_b):
    src = edge_index[0].astype(i32)
    dst = edge_index[1].astype(i32)
    pad_idx = N + (jnp.arange(EPAD - E, dtype=i32) % (NPAD - N))
    src2d = jnp.concatenate([src, pad_idx]).reshape(EROWS, 128)
    dst2d = jnp.concatenate([dst, pad_idx]).reshape(EROWS, 128)
    x_pad = jnp.concatenate([x, jnp.zeros((NPAD - N, 128), f32)], axis=0)

    b0 = conv_b0.reshape(1, 128)
    b1 = conv_b1.reshape(1, 128)
    b2 = conv_b2.reshape(1, 128)
    g0 = bn_g0.reshape(1, 128)
    g1 = bn_g1.reshape(1, 128)
    g2 = bn_g2.reshape(1, 128)
    bb0 = bn_b0.reshape(1, 128)
    bb1 = bn_b1.reshape(1, 128)
    bb2 = bn_b2.reshape(1, 128)
    pw = pool_W.reshape(1, 128)
    pb = pool_b.reshape(1, 1)
    c1a = c1_W[:128]
    c1b = c1_W[128:]
    c1bias = c1_b.reshape(1, 16)
    c2b = c2_b.reshape(1, 32)
    f1b = fc1_b.reshape(1, 128)
    f2b = fc2_b.reshape(1, 2)

    do0, do1, di0, di1 = _sc_deg()(src2d, dst2d)
    h0a5, h0b5, da, db, ia, ib = _call(
        _tc_pre,
        [_sd((NH, 128)), _sd((NH, 128))] + [_sd((NH, 1))] * 4,
        x_pad.reshape(NH, 256), do0.reshape(NH, 32), do1.reshape(NH, 32),
        di0.reshape(NH, 32), di1.reshape(NH, 32))

    agga, aggb = _sc_agg()(h0a5.reshape(NPAD, DH), h0b5.reshape(NPAD, DH),
                           src2d, dst2d)
    h1a5, h1b5 = _call(_tc_layer, [_sd((NH, 128))] * 2,
                       agga.reshape(NH, 128), aggb.reshape(NH, 128),
                       ia, ib, da, db, conv_W0, b0, g0, bb0)

    agga, aggb = _sc_agg()(h1a5.reshape(NPAD, DH), h1b5.reshape(NPAD, DH),
                           src2d, dst2d)
    h2a5, h2b5 = _call(_tc_layer, [_sd((NH, 128))] * 2,
                       agga.reshape(NH, 128), aggb.reshape(NH, 128),
                       ia, ib, da, db, conv_W1, b1, g1, bb1)

    agga, aggb = _sc_agg()(h2a5.reshape(NPAD, DH), h2b5.reshape(NPAD, DH),
                           src2d, dst2d)
    fe, fo, spre = _call(
        _tc_layer3, [_sd((NH, 128)), _sd((NH, 128)), _sd((NH, 32))],
        agga.reshape(NH, 128), aggb.reshape(NH, 128),
        ia, ib, da, db, conv_W2, b2, g2, bb2, pw)

    sca, scb = _sc_score()(spre.reshape(NPAD, 16), src2d, dst2d)
    score = _call(_tc_top, _sd((NH, 2)), sca.reshape(NH, 32),
                  scb.reshape(NH, 32), ia, ib, pb)

    w80, m80 = _call(_tc_sel, [_sd((NPAD // 128, 128))] * 2,
                     score.reshape(NPAD // 128, 128))

    wm = jnp.concatenate([w80.reshape(NH, 2), m80.reshape(NH, 2)], axis=1)
    out = _call(_tc_read, _sd((1, 2)), fe, fo, wm, c1a, c1b, c1bias, c2_W,
                c2b, fc1_W, f1b, fc2_W, f2b)
    return out


# speedup vs baseline: 12.8915x; 1.0124x over previous
import functools

import jax
import jax.numpy as jnp
from jax import lax
from jax.experimental import pallas as pl
from jax.experimental.pallas import tpu as pltpu
from jax.experimental.pallas import tpu_sc as plsc

f32 = jnp.float32
i32 = jnp.int32

N = 10000
NPAD = 10240
E = 320000
EPAD = 327680
EROWS = EPAD // 128
CH_FULL = EROWS // 16
CH_HALF = EROWS // 32
RPT = NPAD // 16
DH = 64
K = 5000

@functools.lru_cache(maxsize=1)
def _mesh():
    return plsc.VectorSubcoreMesh(core_axis_name="c", subcore_axis_name="s")


def _zero_rows(buf, width):
    @pl.loop(0, 128)
    def _(r):
        @pl.loop(0, width, step=16)
        def _(t):
            buf[r, pl.ds(t, 16)] = jnp.zeros((16,), f32)


def _sc_deg_body(src_hbm, dst_hbm, o0_hbm, o1_hbm, i0_hbm, i1_hbm,
                 sidxv, didxv, buf, acco, acci, sem, sem2):
    cid = lax.axis_index("c")
    sid = lax.axis_index("s")
    base = cid * jnp.int32(EROWS // 2) + sid * jnp.int32(CH_HALF)

    pltpu.sync_copy(src_hbm.at[pl.ds(base, CH_HALF)], sidxv)
    pltpu.sync_copy(dst_hbm.at[pl.ds(base, CH_HALF)], didxv)

    _zero_rows(buf, 16)
    @pl.loop(0, RPT // 128)
    def _(r):
        pltpu.sync_copy(buf, acco.at[pl.ds(sid * jnp.int32(RPT) + r * jnp.int32(128), 128)])
        pltpu.sync_copy(buf, acci.at[pl.ds(sid * jnp.int32(RPT) + r * jnp.int32(128), 128)])

    @pl.loop(0, 128)
    def _(r):
        buf[r, pl.ds(0, 16)] = jnp.full((16,), 1.0, f32)

    plsc.subcore_barrier()

    @pl.loop(0, CH_HALF)
    def _(j):
        co = pltpu.async_copy(buf, acco.at[sidxv.at[j]], sem, add=True)
        ci = pltpu.async_copy(buf, acci.at[didxv.at[j]], sem2, add=True)
        co.wait()
        ci.wait()

    plsc.subcore_barrier()

    @pl.when(cid == 0)
    def _():
        pltpu.sync_copy(acco.at[pl.ds(sid * jnp.int32(RPT), RPT)],
                        o0_hbm.at[pl.ds(sid * jnp.int32(RPT), RPT)])
        pltpu.sync_copy(acci.at[pl.ds(sid * jnp.int32(RPT), RPT)],
                        i0_hbm.at[pl.ds(sid * jnp.int32(RPT), RPT)])

    @pl.when(cid == 1)
    def _():
        pltpu.sync_copy(acco.at[pl.ds(sid * jnp.int32(RPT), RPT)],
                        o1_hbm.at[pl.ds(sid * jnp.int32(RPT), RPT)])
        pltpu.sync_copy(acci.at[pl.ds(sid * jnp.int32(RPT), RPT)],
                        i1_hbm.at[pl.ds(sid * jnp.int32(RPT), RPT)])


@functools.lru_cache(maxsize=1)
def _sc_deg():
    return pl.kernel(
        _sc_deg_body,
        mesh=_mesh(),
        out_type=[jax.ShapeDtypeStruct((NPAD, 16), f32)] * 4,
        scratch_types=[
            pltpu.VMEM((CH_HALF, 128), i32),
            pltpu.VMEM((CH_HALF, 128), i32),
            pltpu.VMEM((128, 16), f32),
            pltpu.VMEM_SHARED((NPAD, 16), f32),
            pltpu.VMEM_SHARED((NPAD, 16), f32),
            pltpu.SemaphoreType.DMA,
            pltpu.SemaphoreType.DMA,
        ],
        compiler_params=pltpu.CompilerParams(use_tc_tiling_on_sc=False),
    )


NBUF = 5


def _ring(h_ref, acc, srcv, dstv, bufs, gsem, ssem, nch, rows):
    for b in range(NBUF):
        pltpu.async_copy(h_ref.at[srcv.at[b]], bufs[b], gsem[b])

    @pl.loop(0, nch, step=NBUF)
    def _(j):
        for b in range(NBUF):
            pltpu.make_async_copy(h_ref.at[pl.ds(0, rows)], bufs[b],
                                  gsem[b]).wait()
            pltpu.async_copy(bufs[b], acc.at[dstv.at[j + b]], ssem[b],
                             add=True)
        for b in range(NBUF):
            pltpu.make_async_copy(bufs[b], acc.at[pl.ds(0, rows)],
                                  ssem[b]).wait()

            @pl.when(j + (NBUF + b) < nch)
            def _(b=b):
                pltpu.async_copy(h_ref.at[srcv.at[j + (NBUF + b)]], bufs[b],
                                 gsem[b])


def _sc_agg_body(ha, hb, src_hbm, dst_hbm, outa, outb, srcv, dstv,
                 b0, b1, b2, b3, b4, acc,
                 g0, g1, g2, g3, g4, s0, s1, s2, s3, s4):
    cid = lax.axis_index("c")
    sid = lax.axis_index("s")
    bufs = (b0, b1, b2, b3, b4)
    gsem = (g0, g1, g2, g3, g4)
    ssem = (s0, s1, s2, s3, s4)

    pltpu.sync_copy(src_hbm.at[pl.ds(sid * jnp.int32(CH_FULL), CH_FULL)], srcv)
    pltpu.sync_copy(dst_hbm.at[pl.ds(sid * jnp.int32(CH_FULL), CH_FULL)], dstv)

    _zero_rows(b0, DH)
    @pl.loop(0, RPT // 128)
    def _(r):
        pltpu.sync_copy(b0, acc.at[pl.ds(sid * jnp.int32(RPT) + r * jnp.int32(128), 128)])

    plsc.subcore_barrier()

    @pl.when(cid == 0)
    def _():
        _ring(ha, acc, srcv, dstv, bufs, gsem, ssem, CH_FULL, 128)

    @pl.when(cid == 1)
    def _():
        _ring(hb, acc, srcv, dstv, bufs, gsem, ssem, CH_FULL, 128)

    plsc.subcore_barrier()

    @pl.when(cid == 0)
    def _():
        pltpu.sync_copy(acc.at[pl.ds(sid * jnp.int32(RPT), RPT)],
                        outa.at[pl.ds(sid * jnp.int32(RPT), RPT)])

    @pl.when(cid == 1)
    def _():
        pltpu.sync_copy(acc.at[pl.ds(sid * jnp.int32(RPT), RPT)],
                        outb.at[pl.ds(sid * jnp.int32(RPT), RPT)])


@functools.lru_cache(maxsize=1)
def _sc_agg():
    return pl.kernel(
        _sc_agg_body,
        mesh=_mesh(),
        out_type=[jax.ShapeDtypeStruct((NPAD, DH), f32),
                  jax.ShapeDtypeStruct((NPAD, DH), f32)],
        scratch_types=(
            [pltpu.VMEM((CH_FULL, 128), i32)] * 2
            + [pltpu.VMEM((128, DH), f32)] * NBUF
            + [pltpu.VMEM_SHARED((NPAD, DH), f32)]
            + [pltpu.SemaphoreType.DMA] * (2 * NBUF)
        ),
        compiler_params=pltpu.CompilerParams(use_tc_tiling_on_sc=False),
    )


def _sc_score_body(spre, src_hbm, dst_hbm, outa, outb, srcv, dstv,
                   b0, b1, b2, b3, b4, acc,
                   g0, g1, g2, g3, g4, s0, s1, s2, s3, s4):
    cid = lax.axis_index("c")
    sid = lax.axis_index("s")
    base = cid * jnp.int32(EROWS // 2) + sid * jnp.int32(CH_HALF)
    bufs = (b0, b1, b2, b3, b4)
    gsem = (g0, g1, g2, g3, g4)
    ssem = (s0, s1, s2, s3, s4)

    pltpu.sync_copy(src_hbm.at[pl.ds(base, CH_HALF)], srcv)
    pltpu.sync_copy(dst_hbm.at[pl.ds(base, CH_HALF)], dstv)

    _zero_rows(b0, 16)
    @pl.loop(0, RPT // 128)
    def _(r):
        pltpu.sync_copy(b0, acc.at[pl.ds(sid * jnp.int32(RPT) + r * jnp.int32(128), 128)])

    plsc.subcore_barrier()

    _ring(spre, acc, srcv, dstv, bufs, gsem, ssem, CH_HALF, 128)

    plsc.subcore_barrier()

    @pl.when(cid == 0)
    def _():
        pltpu.sync_copy(acc.at[pl.ds(sid * jnp.int32(RPT), RPT)],
                        outa.at[pl.ds(sid * jnp.int32(RPT), RPT)])

    @pl.when(cid == 1)
    def _():
        pltpu.sync_copy(acc.at[pl.ds(sid * jnp.int32(RPT), RPT)],
                        outb.at[pl.ds(sid * jnp.int32(RPT), RPT)])


@functools.lru_cache(maxsize=1)
def _sc_score():
    return pl.kernel(
        _sc_score_body,
        mesh=_mesh(),
        out_type=[jax.ShapeDtypeStruct((NPAD, 16), f32),
                  jax.ShapeDtypeStruct((NPAD, 16), f32)],
        scratch_types=(
            [pltpu.VMEM((CH_HALF, 128), i32)] * 2
            + [pltpu.VMEM((128, 16), f32)] * NBUF
            + [pltpu.VMEM_SHARED((NPAD, 16), f32)]
            + [pltpu.SemaphoreType.DMA] * (2 * NBUF)
        ),
        compiler_params=pltpu.CompilerParams(use_tc_tiling_on_sc=False),
    )


_HP = dict(preferred_element_type=f32)


NH = NPAD // 2
NMASK = N // 2


def _tc_pre(x_ref, o0_ref, o1_ref, i0_ref, i1_ref, ha_ref, hb_ref,
            da_ref, db_ref, ia_ref, ib_ref):
    dego = o0_ref[...] + o1_ref[...]
    degi = i0_ref[...] + i1_ref[...]
    douta = lax.rsqrt(jnp.maximum(dego[:, 0:1], 1.0))
    doutb = lax.rsqrt(jnp.maximum(dego[:, 16:17], 1.0))
    dina = lax.rsqrt(jnp.maximum(degi[:, 0:1], 1.0))
    dinb = lax.rsqrt(jnp.maximum(degi[:, 16:17], 1.0))
    x5 = x_ref[...]
    he = x5[:, :128] * douta
    ho = x5[:, 128:] * doutb
    ha_ref[...] = jnp.concatenate([he[:, :DH], ho[:, :DH]], axis=1)
    hb_ref[...] = jnp.concatenate([he[:, DH:], ho[:, DH:]], axis=1)
    da_ref[...] = douta
    db_ref[...] = doutb
    ia_ref[...] = dina
    ib_ref[...] = dinb


def _layer_core(agga_ref, aggb_ref, ia_ref, ib_ref, w_ref, b_ref, g_ref,
                bb_ref):
    agga = agga_ref[...]
    aggb = aggb_ref[...]
    even = jnp.concatenate([agga[:, :DH], aggb[:, :DH]], axis=1) * ia_ref[...]
    odd = jnp.concatenate([agga[:, DH:], aggb[:, DH:]], axis=1) * ib_ref[...]
    ze = jnp.dot(even, w_ref[...], **_HP) + b_ref[...]
    zo = jnp.dot(odd, w_ref[...], **_HP) + b_ref[...]
    mask = (lax.broadcasted_iota(i32, (NH, 1), 0) < NMASK).astype(f32)
    zem = ze * mask
    zom = zo * mask
    m = (jnp.sum(zem, axis=0, keepdims=True)
         + jnp.sum(zom, axis=0, keepdims=True)) / N
    ex2 = (jnp.sum(ze * zem, axis=0, keepdims=True)
           + jnp.sum(zo * zom, axis=0, keepdims=True)) / N
    sc = lax.rsqrt(ex2 - m * m + 1e-5)
    fe = jnp.maximum((ze - m) * sc * g_ref[...] + bb_ref[...], 0.0)
    fo = jnp.maximum((zo - m) * sc * g_ref[...] + bb_ref[...], 0.0)
    return fe, fo


def _tc_layer(agga_ref, aggb_ref, ia_ref, ib_ref, da_ref, db_ref, w_ref,
              b_ref, g_ref, bb_ref, ha_ref, hb_ref):
    fe, fo = _layer_core(agga_ref, aggb_ref, ia_ref, ib_ref, w_ref, b_ref,
                         g_ref, bb_ref)
    he = fe * da_ref[...]
    ho = fo * db_ref[...]
    ha_ref[...] = jnp.concatenate([he[:, :DH], ho[:, :DH]], axis=1)
    hb_ref[...] = jnp.concatenate([he[:, DH:], ho[:, DH:]], axis=1)


def _tc_layer3(agga_ref, aggb_ref, ia_ref, ib_ref, da_ref, db_ref, w_ref,
               b_ref, g_ref, bb_ref, pw_ref, fe_ref, fo_ref, spre_ref):
    fe, fo = _layer_core(agga_ref, aggb_ref, ia_ref, ib_ref, w_ref, b_ref,
                         g_ref, bb_ref)
    fe_ref[...] = fe
    fo_ref[...] = fo
    pw = pw_ref[...]
    spre_e = jnp.sum(fe * da_ref[...] * pw, axis=1, keepdims=True)
    spre_o = jnp.sum(fo * db_ref[...] * pw, axis=1, keepdims=True)
    spre_ref[...] = jnp.concatenate(
        [jnp.broadcast_to(spre_e, (NH, 16)),
         jnp.broadcast_to(spre_o, (NH, 16))], axis=1)


def _tc_top(sca_ref, scb_ref, ia_ref, ib_ref, pb_ref, score_ref):
    pb = pb_ref[0, 0]
    se = (sca_ref[:, 0:1] + scb_ref[:, 0:1]) * ia_ref[...] + pb
    so = (sca_ref[:, 16:17] + scb_ref[:, 16:17]) * ib_ref[...] + pb
    real = lax.broadcasted_iota(i32, (NH, 1), 0) < NMASK
    se = jnp.where(real, se, -jnp.inf)
    so = jnp.where(real, so, -jnp.inf)
    score_ref[...] = jnp.concatenate([se, so], axis=1)


def _tc_sel(s_ref, w_ref, m_ref):
    s = s_ref[...]
    b = lax.bitcast_convert_type(s, i32)
    u = jnp.where(b >= 0, b ^ jnp.int32(-2147483648), ~b)

    def vbody(t, carry):
        kk, sel, match = carry
        bit = (31 - t).astype(i32)
        bitv = lax.shift_right_logical(u, bit) & 1
        hi = match * bitv
        c1 = jnp.sum(hi.astype(f32))
        take1 = kk <= c1
        sel = jnp.where(take1, sel, sel | hi)
        kk = jnp.where(take1, kk, kk - c1)
        match = match * jnp.where(take1, bitv, 1 - bitv)
        return kk, sel, match

    init = (jnp.float32(K), jnp.zeros(s.shape, i32),
            jnp.ones(s.shape, i32))
    kk, sel, match = lax.fori_loop(0, 32, vbody, init)

    gidx = (lax.broadcasted_iota(i32, s.shape, 0) * s.shape[1]
            + lax.broadcasted_iota(i32, s.shape, 1))

    def ibody(t, carry):
        kk, sel, match = carry
        bit = (13 - t).astype(i32)
        bitv = lax.shift_right_logical(gidx, bit) & 1
        lo = match * (1 - bitv)
        c0 = jnp.sum(lo.astype(f32))
        take0 = kk <= c0
        sel = jnp.where(take0, sel, sel | lo)
        kk = jnp.where(take0, kk, kk - c0)
        match = match * jnp.where(take0, 1 - bitv, bitv)
        return kk, sel, match

    kk, sel, match = lax.fori_loop(0, 14, ibody, (kk, sel, match))
    sel = sel | match
    mf = sel.astype(f32)
    m_ref[...] = mf
    w_ref[...] = mf * jnp.tanh(s)


def _tc_read(fe_ref, fo_ref, wm_ref, c1a_ref, c1b_ref, c1bias_ref, c2w_ref,
             c2b_ref, f1w_ref, f1b_ref, f2w_ref, f2b_ref, out_ref):
    wm = wm_ref[...]
    fwe = fe_ref[...] * wm[:, 0:1]
    fwo = fo_ref[...] * wm[:, 1:2]
    avg = (jnp.sum(fwe, axis=0, keepdims=True)
           + jnp.sum(fwo, axis=0, keepdims=True)) / K
    mx = jnp.maximum(
        jnp.max(jnp.where(wm[:, 2:3] > 0.5, fwe, -jnp.inf), axis=0,
                keepdims=True),
        jnp.max(jnp.where(wm[:, 3:4] > 0.5, fwo, -jnp.inf), axis=0,
                keepdims=True))
    h = (jnp.dot(avg, c1a_ref[...], **_HP) + jnp.dot(mx, c1b_ref[...], **_HP)
         + c1bias_ref[...])
    h = jnp.maximum(h, 0.0)
    h = jnp.maximum(jnp.dot(h, c2w_ref[...], **_HP) + c2b_ref[...], 0.0)
    h = jnp.maximum(jnp.dot(h, f1w_ref[...], **_HP) + f1b_ref[...], 0.0)
    o = jnp.dot(h, f2w_ref[...], **_HP) + f2b_ref[...]
    omax = jnp.max(o, axis=1, keepdims=True)
    lse = jnp.log(jnp.sum(jnp.exp(o - omax), axis=1, keepdims=True)) + omax
    out_ref[...] = o - lse


def _sd(shape):
    return jax.ShapeDtypeStruct(shape, f32)


def _call(body, out_shapes, *args):
    return pl.pallas_call(body, out_shape=out_shapes)(*args)


def kernel(*args):
    with jax.enable_x64(False):
        return _kernel32(*args)


def _kernel32(x, edge_index, conv_W0, conv_b0, conv_W1, conv_b1, conv_W2,
              conv_b2, bn_g0, bn_b0, bn_g1, bn_b1, bn_g2, bn_b2, pool_W,
              pool_b, c1_W, c1_b, c2_W, c2_b, fc1_W, fc1_b, fc2_W, fc2_b):
    src = edge_index[0].astype(i32)
    dst = edge_index[1].astype(i32)
    pad_idx = N + (jnp.arange(EPAD - E, dtype=i32) % (NPAD - N))
    src2d = jnp.concatenate([src, pad_idx]).reshape(EROWS, 128)
    dst2d = jnp.concatenate([dst, pad_idx]).reshape(EROWS, 128)
    x_pad = jnp.concatenate([x, jnp.zeros((NPAD - N, 128), f32)], axis=0)

    b0 = conv_b0.reshape(1, 128)
    b1 = conv_b1.reshape(1, 128)
    b2 = conv_b2.reshape(1, 128)
    g0 = bn_g0.reshape(1, 128)
    g1 = bn_g1.reshape(1, 128)
    g2 = bn_g2.reshape(1, 128)
    bb0 = bn_b0.reshape(1, 128)
    bb1 = bn_b1.reshape(1, 128)
    bb2 = bn_b2.reshape(1, 128)
    pw = pool_W.reshape(1, 128)
    pb = pool_b.reshape(1, 1)
    c1a = c1_W[:128]
    c1b = c1_W[128:]
    c1bias = c1_b.reshape(1, 16)
    c2b = c2_b.reshape(1, 32)
    f1b = fc1_b.reshape(1, 128)
    f2b = fc2_b.reshape(1, 2)

    do0, do1, di0, di1 = _sc_deg()(src2d, dst2d)
    h0a5, h0b5, da, db, ia, ib = _call(
        _tc_pre,
        [_sd((NH, 128)), _sd((NH, 128))] + [_sd((NH, 1))] * 4,
        x_pad.reshape(NH, 256), do0.reshape(NH, 32), do1.reshape(NH, 32),
        di0.reshape(NH, 32), di1.reshape(NH, 32))

    agga, aggb = _sc_agg()(h0a5.reshape(NPAD, DH), h0b5.reshape(NPAD, DH),
                           src2d, dst2d)
    h1a5, h1b5 = _call(_tc_layer, [_sd((NH, 128))] * 2,
                       agga.reshape(NH, 128), aggb.reshape(NH, 128),
                       ia, ib, da, db, conv_W0, b0, g0, bb0)

    agga, aggb = _sc_agg()(h1a5.reshape(NPAD, DH), h1b5.reshape(NPAD, DH),
                           src2d, dst2d)
    h2a5, h2b5 = _call(_tc_layer, [_sd((NH, 128))] * 2,
                       agga.reshape(NH, 128), aggb.reshape(NH, 128),
                       ia, ib, da, db, conv_W1, b1, g1, bb1)

    agga, aggb = _sc_agg()(h2a5.reshape(NPAD, DH), h2b5.reshape(NPAD, DH),
                           src2d, dst2d)
    fe, fo, spre = _call(
        _tc_layer3, [_sd((NH, 128)), _sd((NH, 128)), _sd((NH, 32))],
        agga.reshape(NH, 128), aggb.reshape(NH, 128),
        ia, ib, da, db, conv_W2, b2, g2, bb2, pw)

    sca, scb = _sc_score()(spre.reshape(NPAD, 16), src2d, dst2d)
    score = _call(_tc_top, _sd((NH, 2)), sca.reshape(NH, 32),
                  scb.reshape(NH, 32), ia, ib, pb)

    w80, m80 = _call(_tc_sel, [_sd((NPAD // 128, 128))] * 2,
                     score.reshape(NPAD // 128, 128))

    wm = jnp.concatenate([w80.reshape(NH, 2), m80.reshape(NH, 2)], axis=1)
    out = _call(_tc_read, _sd((1, 2)), fe, fo, wm, c1a, c1b, c1bias, c2_W,
                c2b, fc1_W, f1b, fc2_W, f2b)
    return out
